# baseline (speedup 1.0000x reference)
import functools

import jax
import jax.numpy as jnp
from jax import lax
from jax.experimental import pallas as pl
from jax.experimental.pallas import tpu as pltpu
from jax.experimental.pallas import tpu_sc as plsc

_GATHER_DNUMS = lax.GatherDimensionNumbers(
    offset_dims=(), collapsed_slice_dims=(0,), start_index_map=(0,))


def _xlane_perm(v, idx16):
    return lax.gather(v, idx16[:, None], _GATHER_DNUMS, (1,),
                      mode=lax.GatherScatterMode.PROMISE_IN_BOUNDS)


def _extract(v, k):
    return jnp.squeeze(lax.slice(v, (k,), (k + 1,)))


ITEM_LEN = 1000000
EMBED_DIM = 64
BATCH = 16384

_NC = 2
_NS = 16
_NW = _NC * _NS
_ROWS_PER_W = BATCH // _NW
_IDX_ROW = 128
_NCHUNK = _ROWS_PER_W // _IDX_ROW


def _sc_body(ti_hbm, cj_hbm, table_hbm, out_hbm,
             ti_v, cj_v, tp_v, cp_v, t_buf, c_buf, out_v, sem):
    wid = lax.axis_index("s") * _NC + lax.axis_index("c")

    base_row = wid * _NCHUNK
    pltpu.sync_copy(ti_hbm.at[pl.ds(base_row, _NCHUNK)], ti_v)
    pltpu.sync_copy(cj_hbm.at[pl.ds(base_row, _NCHUNK)], cj_v)

    def pidx_body(q, carry):
        j = q // 8
        off = (q % 8) * 16
        tp_v[j, pl.ds(off, 16)] = ti_v[j, pl.ds(off, 16)] >> 1
        cp_v[j, pl.ds(off, 16)] = cj_v[j, pl.ds(off, 16)] >> 1
        return carry

    lax.fori_loop(0, _ROWS_PER_W // 16, pidx_body, 0)

    lane = lax.iota(jnp.int32, 16)
    perms = [lane ^ sh for sh in (8, 4, 2, 1)]

    def chunk_body(ch, carry):
        td = pltpu.async_copy(table_hbm.at[tp_v.at[ch]], t_buf, sem)
        cd = pltpu.async_copy(table_hbm.at[cp_v.at[ch]], c_buf, sem)
        td.wait()
        cd.wait()

        def group_body(g, carry2):
            off = g * 16
            t_par = (ti_v[ch, pl.ds(off, 16)] & 1) * EMBED_DIM
            c_par = (cj_v[ch, pl.ds(off, 16)] & 1) * EMBED_DIM
            acc = jnp.zeros((16,), jnp.float32)
            for k in range(16):
                r = off + k
                ts = _extract(t_par, k)
                cs = _extract(c_par, k)
                s = (t_buf[r, pl.ds(ts, 16)] * c_buf[r, pl.ds(cs, 16)]
                     + t_buf[r, pl.ds(ts + 16, 16)]
                     * c_buf[r, pl.ds(cs + 16, 16)]
                     + t_buf[r, pl.ds(ts + 32, 16)]
                     * c_buf[r, pl.ds(cs + 32, 16)]
                     + t_buf[r, pl.ds(ts + 48, 16)]
                     * c_buf[r, pl.ds(cs + 48, 16)])
                for p in perms:
                    s = s + _xlane_perm(s, p)
                acc = jnp.where(lane == k, s, acc)
            out_v[pl.ds(ch * _IDX_ROW + off, 16)] = (
                1.0 / (1.0 + jnp.exp(-acc)))
            return carry2

        lax.fori_loop(0, _IDX_ROW // 16, group_body, 0)
        return carry

    lax.fori_loop(0, _NCHUNK, chunk_body, 0)

    pltpu.sync_copy(out_v, out_hbm.at[pl.ds(wid * _ROWS_PER_W, _ROWS_PER_W)])


@jax.jit
def kernel(target_i, context_j, embedding_table):
    ti = target_i.astype(jnp.int32).reshape(_NW * _NCHUNK, _IDX_ROW)
    cj = context_j.astype(jnp.int32).reshape(_NW * _NCHUNK, _IDX_ROW)
    table2 = embedding_table.reshape(ITEM_LEN // 2, 2 * EMBED_DIM)

    mesh = plsc.VectorSubcoreMesh(core_axis_name="c", subcore_axis_name="s")
    run = functools.partial(
        pl.kernel,
        mesh=mesh,
        out_type=jax.ShapeDtypeStruct((BATCH,), jnp.float32),
        scratch_types=[
            pltpu.VMEM((_NCHUNK, _IDX_ROW), jnp.int32),
            pltpu.VMEM((_NCHUNK, _IDX_ROW), jnp.int32),
            pltpu.VMEM((_NCHUNK, _IDX_ROW), jnp.int32),
            pltpu.VMEM((_NCHUNK, _IDX_ROW), jnp.int32),
            pltpu.VMEM((_IDX_ROW, 2 * EMBED_DIM), jnp.float32),
            pltpu.VMEM((_IDX_ROW, 2 * EMBED_DIM), jnp.float32),
            pltpu.VMEM((_ROWS_PER_W,), jnp.float32),
            pltpu.SemaphoreType.DMA,
        ],
        compiler_params=pltpu.CompilerParams(skip_device_barrier=True),
    )(_sc_body)
    return run(ti, cj, table2)


# speedup vs baseline: 1.6923x; 1.6923x over previous
import functools

import jax
import jax.numpy as jnp
from jax import lax
from jax.experimental import pallas as pl
from jax.experimental.pallas import tpu as pltpu
from jax.experimental.pallas import tpu_sc as plsc

_GATHER_DNUMS = lax.GatherDimensionNumbers(
    offset_dims=(), collapsed_slice_dims=(0,), start_index_map=(0,))


def _xlane_perm(v, idx16):
    return lax.gather(v, idx16[:, None], _GATHER_DNUMS, (1,),
                      mode=lax.GatherScatterMode.PROMISE_IN_BOUNDS)


def _extract(v, k):
    return jnp.squeeze(lax.slice(v, (k,), (k + 1,)))


ITEM_LEN = 1000000
EMBED_DIM = 64
BATCH = 16384

_NC = 2
_NS = 16
_NW = _NC * _NS
_ROWS_PER_W = BATCH // _NW
_CHUNK = 16
_NCHUNK = _ROWS_PER_W // _CHUNK
_IDX_ROW = 128
_IDX_ROWS_PER_W = _ROWS_PER_W // _IDX_ROW


def _sc_body(ti_hbm, cj_hbm, table_hbm, out_hbm,
             ti_v, cj_v, t_flat, c_flat, t_flat2, c_flat2, out_v,
             sem, sem2):
    wid = lax.axis_index("s") * _NC + lax.axis_index("c")

    base_row = wid * _IDX_ROWS_PER_W
    pltpu.sync_copy(ti_hbm.at[pl.ds(base_row, _IDX_ROWS_PER_W)], ti_v)
    pltpu.sync_copy(cj_hbm.at[pl.ds(base_row, _IDX_ROWS_PER_W)], cj_v)

    lane = lax.iota(jnp.int32, 16)
    perms = [lane ^ sh for sh in (8, 4, 2, 1)]

    def fire(ch, t_buf, c_buf, s):
        t_idx = ti_v[ch // 8, pl.ds((ch % 8) * _CHUNK, _CHUNK)]
        c_idx = cj_v[ch // 8, pl.ds((ch % 8) * _CHUNK, _CHUNK)]
        for k in range(_CHUNK):
            pltpu.async_copy(table_hbm.at[_extract(t_idx, k)],
                             t_buf.at[k], s)
            pltpu.async_copy(table_hbm.at[_extract(c_idx, k)],
                             c_buf.at[k], s)

    def drain(t_buf, c_buf, s):
        for k in range(_CHUNK):
            pltpu.make_async_copy(table_hbm.at[0], t_buf.at[k], s).wait()
            pltpu.make_async_copy(table_hbm.at[0], c_buf.at[k], s).wait()

    def compute(ch, t_buf, c_buf):
        acc = jnp.zeros((16,), jnp.float32)
        for k in range(_CHUNK):
            s = (t_buf[k, pl.ds(0, 16)] * c_buf[k, pl.ds(0, 16)]
                 + t_buf[k, pl.ds(16, 16)] * c_buf[k, pl.ds(16, 16)]
                 + t_buf[k, pl.ds(32, 16)] * c_buf[k, pl.ds(32, 16)]
                 + t_buf[k, pl.ds(48, 16)] * c_buf[k, pl.ds(48, 16)])
            for p in perms:
                s = s + _xlane_perm(s, p)
            acc = jnp.where(lane == k, s, acc)
        out_v[pl.ds(ch * _CHUNK, _CHUNK)] = 1.0 / (1.0 + jnp.exp(-acc))

    fire(0, t_flat, c_flat, sem)

    def step(st, carry):
        ch0 = 2 * st
        fire(ch0 + 1, t_flat2, c_flat2, sem2)
        drain(t_flat, c_flat, sem)
        compute(ch0, t_flat, c_flat)

        @pl.when(ch0 + 2 < _NCHUNK)
        def _():
            fire(ch0 + 2, t_flat, c_flat, sem)

        drain(t_flat2, c_flat2, sem2)
        compute(ch0 + 1, t_flat2, c_flat2)
        return carry

    lax.fori_loop(0, _NCHUNK // 2, step, 0)

    pltpu.sync_copy(out_v, out_hbm.at[pl.ds(wid * _ROWS_PER_W, _ROWS_PER_W)])


@jax.jit
def kernel(target_i, context_j, embedding_table):
    ti = target_i.astype(jnp.int32).reshape(_NW * _IDX_ROWS_PER_W, _IDX_ROW)
    cj = context_j.astype(jnp.int32).reshape(_NW * _IDX_ROWS_PER_W, _IDX_ROW)

    mesh = plsc.VectorSubcoreMesh(core_axis_name="c", subcore_axis_name="s")
    run = functools.partial(
        pl.kernel,
        mesh=mesh,
        out_type=jax.ShapeDtypeStruct((BATCH,), jnp.float32),
        scratch_types=[
            pltpu.VMEM((_IDX_ROWS_PER_W, _IDX_ROW), jnp.int32),
            pltpu.VMEM((_IDX_ROWS_PER_W, _IDX_ROW), jnp.int32),
            pltpu.VMEM((_CHUNK, EMBED_DIM), jnp.float32),
            pltpu.VMEM((_CHUNK, EMBED_DIM), jnp.float32),
            pltpu.VMEM((_CHUNK, EMBED_DIM), jnp.float32),
            pltpu.VMEM((_CHUNK, EMBED_DIM), jnp.float32),
            pltpu.VMEM((_ROWS_PER_W,), jnp.float32),
            pltpu.SemaphoreType.DMA,
            pltpu.SemaphoreType.DMA,
        ],
        compiler_params=pltpu.CompilerParams(skip_device_barrier=True),
    )(_sc_body)
    return run(ti, cj, embedding_table)


# speedup vs baseline: 2.4993x; 1.4769x over previous
import functools

import jax
import jax.numpy as jnp
from jax import lax
from jax.experimental import pallas as pl
from jax.experimental.pallas import tpu as pltpu
from jax.experimental.pallas import tpu_sc as plsc

_GATHER_DNUMS = lax.GatherDimensionNumbers(
    offset_dims=(), collapsed_slice_dims=(0,), start_index_map=(0,))


def _xlane_perm(v, idx16):
    return lax.gather(v, idx16[:, None], _GATHER_DNUMS, (1,),
                      mode=lax.GatherScatterMode.PROMISE_IN_BOUNDS)


def _extract(v, k):
    return jnp.squeeze(lax.slice(v, (k,), (k + 1,)))


ITEM_LEN = 1000000
EMBED_DIM = 64
BATCH = 16384

_NC = 2
_NS = 16
_NW = _NC * _NS
_ROWS_PER_W = BATCH // _NW
_CHUNK = 16
_NCHUNK = _ROWS_PER_W // _CHUNK
_IDX_ROW = 128
_IDX_ROWS_PER_W = _ROWS_PER_W // _IDX_ROW


def _sc_body(ti_hbm, cj_hbm, table_hbm, out_hbm,
             ti_v, cj_v, t_flat, c_flat, t_flat2, c_flat2, out_v,
             sem, sem2):
    wid = lax.axis_index("s") * _NC + lax.axis_index("c")

    base_row = wid * _IDX_ROWS_PER_W
    pltpu.sync_copy(ti_hbm.at[pl.ds(base_row, _IDX_ROWS_PER_W)], ti_v)
    pltpu.sync_copy(cj_hbm.at[pl.ds(base_row, _IDX_ROWS_PER_W)], cj_v)

    lane = lax.iota(jnp.int32, 16)
    perms = [lane ^ sh for sh in (8, 4, 2, 1)]

    def fire(ch, t_buf, c_buf, s):
        t_idx = ti_v[ch // 8, pl.ds((ch % 8) * _CHUNK, _CHUNK)]
        c_idx = cj_v[ch // 8, pl.ds((ch % 8) * _CHUNK, _CHUNK)]
        for k in range(_CHUNK):
            tk = _extract(t_idx, k)
            ck = _extract(c_idx, k)
            pltpu.async_copy(table_hbm.at[tk >> 3, tk & 7],
                             t_buf.at[k], s)
            pltpu.async_copy(table_hbm.at[ck >> 3, ck & 7],
                             c_buf.at[k], s)

    def drain(t_buf, c_buf, s):
        for k in range(_CHUNK):
            pltpu.make_async_copy(table_hbm.at[0, 0], t_buf.at[k], s).wait()
            pltpu.make_async_copy(table_hbm.at[0, 0], c_buf.at[k], s).wait()

    def compute(ch, t_buf, c_buf):
        acc = jnp.zeros((16,), jnp.float32)
        for k in range(_CHUNK):
            s = (t_buf[k, pl.ds(0, 16)] * c_buf[k, pl.ds(0, 16)]
                 + t_buf[k, pl.ds(16, 16)] * c_buf[k, pl.ds(16, 16)]
                 + t_buf[k, pl.ds(32, 16)] * c_buf[k, pl.ds(32, 16)]
                 + t_buf[k, pl.ds(48, 16)] * c_buf[k, pl.ds(48, 16)])
            for p in perms:
                s = s + _xlane_perm(s, p)
            acc = jnp.where(lane == k, s, acc)
        out_v[pl.ds(ch * _CHUNK, _CHUNK)] = 1.0 / (1.0 + jnp.exp(-acc))

    fire(0, t_flat, c_flat, sem)

    def step(st, carry):
        ch0 = 2 * st
        fire(ch0 + 1, t_flat2, c_flat2, sem2)
        drain(t_flat, c_flat, sem)
        compute(ch0, t_flat, c_flat)

        @pl.when(ch0 + 2 < _NCHUNK)
        def _():
            fire(ch0 + 2, t_flat, c_flat, sem)

        drain(t_flat2, c_flat2, sem2)
        compute(ch0 + 1, t_flat2, c_flat2)
        return carry

    lax.fori_loop(0, _NCHUNK // 2, step, 0)

    pltpu.sync_copy(out_v, out_hbm.at[pl.ds(wid * _ROWS_PER_W, _ROWS_PER_W)])


@jax.jit
def kernel(target_i, context_j, embedding_table):
    ti = target_i.astype(jnp.int32).reshape(_NW * _IDX_ROWS_PER_W, _IDX_ROW)
    cj = context_j.astype(jnp.int32).reshape(_NW * _IDX_ROWS_PER_W, _IDX_ROW)

    mesh = plsc.VectorSubcoreMesh(core_axis_name="c", subcore_axis_name="s")
    run = functools.partial(
        pl.kernel,
        mesh=mesh,
        out_type=jax.ShapeDtypeStruct((BATCH,), jnp.float32),
        scratch_types=[
            pltpu.VMEM((_IDX_ROWS_PER_W, _IDX_ROW), jnp.int32),
            pltpu.VMEM((_IDX_ROWS_PER_W, _IDX_ROW), jnp.int32),
            pltpu.VMEM((_CHUNK, EMBED_DIM), jnp.float32),
            pltpu.VMEM((_CHUNK, EMBED_DIM), jnp.float32),
            pltpu.VMEM((_CHUNK, EMBED_DIM), jnp.float32),
            pltpu.VMEM((_CHUNK, EMBED_DIM), jnp.float32),
            pltpu.VMEM((_ROWS_PER_W,), jnp.float32),
            pltpu.SemaphoreType.DMA,
            pltpu.SemaphoreType.DMA,
        ],
        compiler_params=pltpu.CompilerParams(skip_device_barrier=True),
    )(_sc_body)
    return run(ti, cj, embedding_table.reshape(ITEM_LEN // 8, 8, EMBED_DIM))
